# Initial kernel scaffold; baseline (speedup 1.0000x reference)
import jax
import jax.numpy as jnp
from jax.experimental import pallas as pl


def kernel(z, pos, batch, edge_index, params):
    raise NotImplementedError("write your pallas kernel here")


# speedup vs baseline: 2.5127x; 2.5127x over previous
import functools

import jax
import jax.numpy as jnp
import numpy as np
from jax import lax
from jax.experimental import pallas as pl
from jax.experimental.pallas import tpu as pltpu
from jax.experimental.pallas import tpu_sc as plsc

N = 50000
E = 800000
B = 64
NR = 16
REMB = 32
RHID = 64
CUTOFF = 5.0

F32 = jnp.float32
I32 = jnp.int32

_LANES = 16
_CHUNK = 128
_NTILES = 32
_NROWS = 50176
_RP = _NROWS // 16
_ZROWS = 392
_NP = 50176

_SH0 = float(1.0 / (2.0 * np.sqrt(np.pi)))
_C1 = float(np.sqrt(3.0 / (4.0 * np.pi)))
_INV_SQRT3 = float(1.0 / np.sqrt(3.0))
_INV_SQRT2 = float(1.0 / np.sqrt(2.0))
_INV_SQRT_NR = float(1.0 / np.sqrt(NR))


def _silu(x):
    return x * jax.nn.sigmoid(x)


def _gather_rows(table, idx, d):
    m = idx.shape[0]
    nchunks = m // _CHUNK
    per = (nchunks + _NTILES - 1) // _NTILES
    mesh = plsc.VectorSubcoreMesh(core_axis_name="c", subcore_axis_name="s")

    @functools.partial(
        pl.kernel,
        out_type=jax.ShapeDtypeStruct((m, d), F32),
        mesh=mesh,
        compiler_params=pltpu.CompilerParams(use_tc_tiling_on_sc=False),
        scratch_types=[
            pltpu.VMEM((_CHUNK,), I32),
            pltpu.VMEM((_CHUNK, d), F32),
            pltpu.SemaphoreType.DMA,
        ],
    )
    def k(table_hbm, idx_hbm, out_hbm, idx_v, rows_v, sem):
        wid = lax.axis_index("c") * 16 + lax.axis_index("s")

        def body(i, carry):
            chunk = wid * per + i

            @pl.when(chunk < nchunks)
            def _():
                base = chunk * _CHUNK
                pltpu.sync_copy(idx_hbm.at[pl.ds(base, _CHUNK)], idx_v)
                pltpu.async_copy(table_hbm.at[idx_v], rows_v, sem).wait()
                pltpu.sync_copy(rows_v, out_hbm.at[pl.ds(base, _CHUNK)])

            return carry

        lax.fori_loop(0, per, body, 0)

    return k(table, idx)


def _scatter_partials(dst, msg, dtot):
    dp = 32 if dtot >= 32 else dtot
    npass = dtot // dp
    nchunks = E // _CHUNK
    per = (nchunks + _NTILES - 1) // _NTILES
    mesh = plsc.VectorSubcoreMesh(core_axis_name="c", subcore_axis_name="s")

    @functools.partial(
        pl.kernel,
        out_type=jax.ShapeDtypeStruct((2, _NROWS, dtot), F32),
        mesh=mesh,
        compiler_params=pltpu.CompilerParams(use_tc_tiling_on_sc=False),
        scratch_types=[
            pltpu.VMEM_SHARED((_NROWS, dp), F32),
            pltpu.VMEM((_CHUNK,), I32),
            pltpu.VMEM((_CHUNK, dp), F32),
            pltpu.VMEM((_ZROWS, dp), F32),
        ],
    )
    def k(dst_hbm, msg_hbm, out_hbm, acc, idx_v, val_v, zbuf):
        cid = lax.axis_index("c")
        sid = lax.axis_index("s")
        wid = cid * 16 + sid

        def zrow(i, carry):
            for j in range(dp // _LANES):
                zbuf[i, pl.ds(j * _LANES, _LANES)] = jnp.zeros((_LANES,), F32)
            return carry

        lax.fori_loop(0, _ZROWS, zrow, 0)

        for p in range(npass):
            def zfill(i, carry):
                pltpu.sync_copy(
                    zbuf, acc.at[pl.ds(sid * _RP + i * _ZROWS, _ZROWS)])
                return carry

            lax.fori_loop(0, _RP // _ZROWS, zfill, 0)
            plsc.subcore_barrier()

            def body(i, carry):
                chunk = wid * per + i

                @pl.when(chunk < nchunks)
                def _():
                    base = chunk * _CHUNK
                    pltpu.sync_copy(dst_hbm.at[pl.ds(base, _CHUNK)], idx_v)
                    pltpu.sync_copy(
                        msg_hbm.at[pl.ds(base, _CHUNK), pl.ds(p * dp, dp)],
                        val_v)
                    pltpu.sync_copy(val_v, acc.at[idx_v], add=True)

                return carry

            lax.fori_loop(0, per, body, 0)
            plsc.subcore_barrier()

            pltpu.sync_copy(
                acc.at[pl.ds(sid * _RP, _RP)],
                out_hbm.at[cid, pl.ds(sid * _RP, _RP), pl.ds(p * dp, dp)])
            plsc.subcore_barrier()

    return k(dst, msg)


_BE = 4000
_BN = 2000

_CENTERS = np.linspace(0.0, CUTOFF, REMB).astype(np.float32)
_RBF_STEP = float(_CENTERS[1] - _CENTERS[0])
_RBF_SCALE = float(np.sqrt(REMB) / 1.12)


def _rbf(length):
    c = lax.broadcasted_iota(I32, (1, REMB), 1).astype(F32) * _RBF_STEP
    diff = (length - c) * (1.0 / _RBF_STEP)
    return jnp.exp(-diff * diff) * _RBF_SCALE


def _radial_mlp(emb, w1, b1, w2, b2, w3, b3):
    h = _silu(jnp.dot(emb, w1, preferred_element_type=F32) + b1)
    h = _silu(jnp.dot(h, w2, preferred_element_type=F32) + b2)
    return jnp.dot(h, w3, preferred_element_type=F32) + b3


def _full(shape):
    return pl.BlockSpec(shape, lambda i: (0, 0))


def _rows(bs, cols):
    return pl.BlockSpec((bs, cols), lambda i: (i, 0))


def _tc_geom(ps, pd):
    def body(ps_ref, pd_ref, geom_ref):
        ex = ps_ref[:, 0:1] - pd_ref[:, 0:1]
        ey = ps_ref[:, 1:2] - pd_ref[:, 1:2]
        ez = ps_ref[:, 2:3] - pd_ref[:, 2:3]
        length = jnp.sqrt(ex * ex + ey * ey + ez * ez) + 1e-8
        inv = _C1 / length
        geom_ref[...] = jnp.concatenate(
            [ey * inv, ez * inv, ex * inv, length], axis=1)

    return pl.pallas_call(
        body,
        grid=(E // _BE,),
        in_specs=[_rows(_BE, 16), _rows(_BE, 16)],
        out_specs=_rows(_BE, 4),
        out_shape=jax.ShapeDtypeStruct((E, 4), F32),
    )(ps, pd)


def _edge_weights(p, l):
    return (p['l%d_W1' % l], p['l%d_b1' % l].reshape(1, RHID),
            p['l%d_W2' % l], p['l%d_b2' % l].reshape(1, REMB),
            p['l%d_W3' % l], p['l%d_b3' % l].reshape(1, -1))


def _tc_edge0(geom, fsrc, p):
    nu = 2 * NR

    def body(g_ref, f_ref, w1, b1, w2, b2, w3, b3, msg_ref):
        emb = _rbf(g_ref[:, 3:4])
        w = _radial_mlp(emb, w1[...], b1[...], w2[...], b2[...], w3[...],
                        b3[...])
        f = f_ref[...]
        w0f = w[:, 0:NR] * f * _SH0
        w1f = w[:, NR:2 * NR] * f
        msg_ref[...] = jnp.concatenate(
            [w0f, w1f * g_ref[:, 0:1], w1f * g_ref[:, 1:2],
             w1f * g_ref[:, 2:3]], axis=1)

    return pl.pallas_call(
        body,
        grid=(E // _BE,),
        in_specs=[
            _rows(_BE, 4), _rows(_BE, NR),
            _full((REMB, RHID)), _full((1, RHID)),
            _full((RHID, REMB)), _full((1, REMB)),
            _full((REMB, nu)), _full((1, nu)),
        ],
        out_specs=_rows(_BE, 4 * NR),
        out_shape=jax.ShapeDtypeStruct((E, 4 * NR), F32),
    )(geom, fsrc, *_edge_weights(p, 0))


def _tc_edge1(geom, srcfeat, p):
    nu = 5 * NR

    def body(g_ref, x_ref, w1, b1, w2, b2, w3, b3, msg_ref):
        emb = _rbf(g_ref[:, 3:4])
        w = _radial_mlp(emb, w1[...], b1[...], w2[...], b2[...], w3[...],
                        b3[...])
        s = x_ref[:, 0:NR]
        vx = x_ref[:, NR:2 * NR]
        vy = x_ref[:, 2 * NR:3 * NR]
        vz = x_ref[:, 3 * NR:4 * NR]
        b0 = g_ref[:, 0:1]
        b1g = g_ref[:, 1:2]
        b2g = g_ref[:, 2:3]
        w0 = w[:, 0:NR]
        w1v = w[:, NR:2 * NR]
        w2v = w[:, 2 * NR:3 * NR]
        w3v = w[:, 3 * NR:4 * NR]
        w4v = w[:, 4 * NR:5 * NR]
        vdot = vx * b0 + vy * b1g + vz * b2g
        es = (w0 * s * _SH0 + w3v * vdot * _INV_SQRT3) * _INV_SQRT2
        cx = vy * b2g - vz * b1g
        cy = vz * b0 - vx * b2g
        cz = vx * b1g - vy * b0
        w1s = w1v * s
        evx = (w1s * b0 + w2v * vx * _SH0 + w4v * cx * _INV_SQRT2) * _INV_SQRT3
        evy = (w1s * b1g + w2v * vy * _SH0 + w4v * cy * _INV_SQRT2) * _INV_SQRT3
        evz = (w1s * b2g + w2v * vz * _SH0 + w4v * cz * _INV_SQRT2) * _INV_SQRT3
        msg_ref[...] = jnp.concatenate([es, evx, evy, evz], axis=1)

    return pl.pallas_call(
        body,
        grid=(E // _BE,),
        in_specs=[
            _rows(_BE, 4), _rows(_BE, 4 * NR),
            _full((REMB, RHID)), _full((1, RHID)),
            _full((RHID, REMB)), _full((1, REMB)),
            _full((REMB, nu)), _full((1, nu)),
        ],
        out_specs=_rows(_BE, 4 * NR),
        out_shape=jax.ShapeDtypeStruct((E, 4 * NR), F32),
    )(geom, srcfeat, *_edge_weights(p, 1))


def _tc_edge2(geom, srcfeat, p):
    nu = 2 * NR

    def body(g_ref, x_ref, w1, b1, w2, b2, w3, b3, msg_ref):
        emb = _rbf(g_ref[:, 3:4])
        w = _radial_mlp(emb, w1[...], b1[...], w2[...], b2[...], w3[...],
                        b3[...])
        s = x_ref[:, 0:NR]
        vx = x_ref[:, NR:2 * NR]
        vy = x_ref[:, 2 * NR:3 * NR]
        vz = x_ref[:, 3 * NR:4 * NR]
        vdot = (vx * g_ref[:, 0:1] + vy * g_ref[:, 1:2] + vz * g_ref[:, 2:3])
        es = (w[:, 0:NR] * s * _SH0
              + w[:, NR:2 * NR] * vdot * _INV_SQRT3) * _INV_SQRT2
        msg_ref[...] = es

    return pl.pallas_call(
        body,
        grid=(E // _BE,),
        in_specs=[
            _rows(_BE, 4), _rows(_BE, 4 * NR),
            _full((REMB, RHID)), _full((1, RHID)),
            _full((RHID, REMB)), _full((1, REMB)),
            _full((REMB, nu)), _full((1, nu)),
        ],
        out_specs=_rows(_BE, NR),
        out_shape=jax.ShapeDtypeStruct((E, NR), F32),
    )(geom, srcfeat, *_edge_weights(p, 2))


def _norm_act(s, vx, vy, vz):
    nrm = jnp.sqrt((vx * vx + vy * vy + vz * vz) * _INV_SQRT3 + 1e-8)
    gate = jax.nn.sigmoid(nrm)
    return _silu(s), vx * gate, vy * gate, vz * gate


def _tc_node0(p0, p1, feat, sc_s):
    def body(p0_ref, p1_ref, f_ref, sc_ref, out_ref):
        s = (p0_ref[:, 0:NR] + p1_ref[:, 0:NR]
             + jnp.dot(f_ref[...], sc_ref[...],
                       preferred_element_type=F32) * _INV_SQRT_NR)
        vx = p0_ref[:, NR:2 * NR] + p1_ref[:, NR:2 * NR]
        vy = p0_ref[:, 2 * NR:3 * NR] + p1_ref[:, 2 * NR:3 * NR]
        vz = p0_ref[:, 3 * NR:4 * NR] + p1_ref[:, 3 * NR:4 * NR]
        s, vx, vy, vz = _norm_act(s, vx, vy, vz)
        out_ref[...] = jnp.concatenate([s, vx, vy, vz], axis=1)

    return pl.pallas_call(
        body,
        grid=(N // _BN,),
        in_specs=[_rows(_BN, 4 * NR), _rows(_BN, 4 * NR), _rows(_BN, NR),
                  _full((NR, NR))],
        out_specs=_rows(_BN, 4 * NR),
        out_shape=jax.ShapeDtypeStruct((N, 4 * NR), F32),
    )(p0, p1, feat, sc_s)


def _tc_node1(p0, p1, prev, sc_s, sc_v):
    def body(p0_ref, p1_ref, x_ref, scs_ref, scv_ref, out_ref):
        scs = scs_ref[...]
        scv = scv_ref[...]
        s = (p0_ref[:, 0:NR] + p1_ref[:, 0:NR]
             + jnp.dot(x_ref[:, 0:NR], scs,
                       preferred_element_type=F32) * _INV_SQRT_NR)
        vx = (p0_ref[:, NR:2 * NR] + p1_ref[:, NR:2 * NR]
              + jnp.dot(x_ref[:, NR:2 * NR], scv,
                        preferred_element_type=F32) * _INV_SQRT_NR)
        vy = (p0_ref[:, 2 * NR:3 * NR] + p1_ref[:, 2 * NR:3 * NR]
              + jnp.dot(x_ref[:, 2 * NR:3 * NR], scv,
                        preferred_element_type=F32) * _INV_SQRT_NR)
        vz = (p0_ref[:, 3 * NR:4 * NR] + p1_ref[:, 3 * NR:4 * NR]
              + jnp.dot(x_ref[:, 3 * NR:4 * NR], scv,
                        preferred_element_type=F32) * _INV_SQRT_NR)
        s, vx, vy, vz = _norm_act(s, vx, vy, vz)
        out_ref[...] = jnp.concatenate([s, vx, vy, vz], axis=1)

    return pl.pallas_call(
        body,
        grid=(N // _BN,),
        in_specs=[_rows(_BN, 4 * NR), _rows(_BN, 4 * NR), _rows(_BN, 4 * NR),
                  _full((NR, NR)), _full((NR, NR))],
        out_specs=_rows(_BN, 4 * NR),
        out_shape=jax.ShapeDtypeStruct((N, 4 * NR), F32),
    )(p0, p1, prev, sc_s, sc_v)


def _tc_node2_pool(p0, p1, prev, sc_s, batch2d):
    def body(p0_ref, p1_ref, x_ref, sc_ref, b_ref, out_ref):
        s = (p0_ref[...] + p1_ref[...]
             + jnp.dot(x_ref[:, 0:NR], sc_ref[...],
                       preferred_element_type=F32) * _INV_SQRT_NR)
        iota = lax.broadcasted_iota(I32, (1, B), 1)
        onehot = (b_ref[...] == iota).astype(F32)
        contrib = lax.dot_general(
            onehot, s, (((0,), (0,)), ((), ())), preferred_element_type=F32)
        i = pl.program_id(0)

        @pl.when(i == 0)
        def _():
            out_ref[...] = contrib

        @pl.when(i != 0)
        def _():
            out_ref[...] = out_ref[...] + contrib

    return pl.pallas_call(
        body,
        grid=(N // _BN,),
        in_specs=[_rows(_BN, NR), _rows(_BN, NR), _rows(_BN, 4 * NR),
                  _full((NR, NR)), _rows(_BN, 1)],
        out_specs=pl.BlockSpec((B, NR), lambda i: (0, 0)),
        out_shape=jax.ShapeDtypeStruct((B, NR), F32),
    )(p0, p1, prev, sc_s, batch2d)


def _tc_head(pooled, p):
    def body(x_ref, w1, b1, w2, b2, w3, b3, out_ref):
        h = _silu(jnp.dot(x_ref[...], w1[...],
                          preferred_element_type=F32) + b1[...])
        h = _silu(jnp.dot(h, w2[...], preferred_element_type=F32) + b2[...])
        o = jnp.dot(h, w3[...], preferred_element_type=F32) + b3[...]
        out_ref[...] = jax.nn.sigmoid(o)

    return pl.pallas_call(
        body,
        grid=(1,),
        in_specs=[
            _full((B, NR)),
            _full((NR, 2 * NR)), _full((1, 2 * NR)),
            _full((2 * NR, NR)), _full((1, NR)),
            _full((NR, 1)), _full((1, 1)),
        ],
        out_specs=_full((B, 1)),
        out_shape=jax.ShapeDtypeStruct((B, 1), F32),
    )(pooled, p['f_W1'], p['f_b1'].reshape(1, 2 * NR),
      p['f_W2'], p['f_b2'].reshape(1, NR),
      p['f_W3'], p['f_b3'].reshape(1, 1))


def kernel(z, pos, batch, edge_index, params):
    p = params
    src = edge_index[0].astype(I32)
    dst = edge_index[1].astype(I32)

    zpad = jnp.concatenate(
        [z.astype(I32), jnp.zeros((_NP - N,), I32)])
    feat = _gather_rows(p['emb'].astype(F32), zpad, NR)[:N]

    pos16 = jnp.zeros((N, 16), F32).at[:, 0:3].set(pos.astype(F32))
    ps = _gather_rows(pos16, src, 16)
    pd = _gather_rows(pos16, dst, 16)
    geom = _tc_geom(ps, pd)

    fsrc = _gather_rows(feat, src, NR)
    msg0 = _tc_edge0(geom, fsrc, p)
    parts0 = _scatter_partials(dst, msg0, 4 * NR)
    packed1 = _tc_node0(parts0[0, :N], parts0[1, :N], feat, p['l0_sc_s'])

    src1 = _gather_rows(packed1, src, 4 * NR)
    msg1 = _tc_edge1(geom, src1, p)
    parts1 = _scatter_partials(dst, msg1, 4 * NR)
    packed2 = _tc_node1(parts1[0, :N], parts1[1, :N], packed1,
                        p['l1_sc_s'], p['l1_sc_v'])

    src2 = _gather_rows(packed2, src, 4 * NR)
    msg2 = _tc_edge2(geom, src2, p)
    parts2 = _scatter_partials(dst, msg2, NR)
    pooled = _tc_node2_pool(parts2[0, :N], parts2[1, :N], packed2,
                            p['l2_sc_s'], batch.astype(I32).reshape(N, 1))

    out = _tc_head(pooled, p)
    return out[:, 0]
